# baseline (speedup 1.0000x reference)
import functools

import jax
import jax.numpy as jnp
from jax import lax
from jax.experimental import pallas as pl
from jax.experimental.pallas import tpu as pltpu
from jax.experimental.pallas import tpu_sc as plsc

NUM_EMB = 1000000
D = 32
B = 16384 * 50
NC, NS = 2, 16
NW = NC * NS
B_PER_W = B // NW
CHUNK = 1600
N_CHUNKS = B_PER_W // CHUNK

_mesh = plsc.VectorSubcoreMesh(core_axis_name="c", subcore_axis_name="s")


@functools.partial(
    pl.kernel,
    mesh=_mesh,
    out_type=jax.ShapeDtypeStruct((B, D), jnp.float32),
    scratch_types=[
        pltpu.VMEM((N_CHUNKS, CHUNK), jnp.int32),
        pltpu.VMEM((CHUNK, D), jnp.float32),
        pltpu.VMEM((CHUNK, D), jnp.float32),
        pltpu.SemaphoreType.DMA,
        pltpu.SemaphoreType.DMA,
        pltpu.SemaphoreType.DMA,
    ],
    compiler_params=pltpu.CompilerParams(use_tc_tiling_on_sc=False),
)
def _gather_kernel(idx_hbm, table_hbm, out_hbm, idx_v, rows0, rows1,
                   sem_g, sem_w0, sem_w1):
    wid = lax.axis_index("s") * NC + lax.axis_index("c")
    base = wid * B_PER_W

    rows = (rows0, rows1)
    sem_w = (sem_w0, sem_w1)

    pltpu.sync_copy(idx_hbm.at[wid], idx_v)

    wb = [None, None]
    for j in range(N_CHUNKS):
        b = j % 2
        if j >= 2:
            wb[b].wait()
        pltpu.async_copy(table_hbm.at[idx_v.at[j]], rows[b], sem_g).wait()
        wb[b] = pltpu.async_copy(
            rows[b], out_hbm.at[pl.ds(base + j * CHUNK, CHUNK)], sem_w[b])
    wb[0].wait()
    wb[1].wait()


def kernel(action, emb_table):
    idx = action.reshape(NW, N_CHUNKS, CHUNK)
    out = _gather_kernel(idx, emb_table)
    return out.reshape(action.shape + (D,))


# speedup vs baseline: 1.3527x; 1.3527x over previous
import functools

import jax
import jax.numpy as jnp
from jax import lax
from jax.experimental import pallas as pl
from jax.experimental.pallas import tpu as pltpu
from jax.experimental.pallas import tpu_sc as plsc

E = 1000000
D = 32
BROWS = 16384
S = 50
NC, NS = 2, 16
NW = NC * NS
BW = BROWS // NW

_mesh = plsc.VectorSubcoreMesh(core_axis_name="c", subcore_axis_name="s")


@functools.partial(
    pl.kernel,
    mesh=_mesh,
    out_type=jax.ShapeDtypeStruct((S, D, BROWS), jnp.float32),
    scratch_types=[
        pltpu.VMEM((S, BW), jnp.int32),
        pltpu.VMEM((BW, D), jnp.float32),
        pltpu.VMEM((BW, D), jnp.float32),
        pltpu.VMEM((D, BW), jnp.float32),
        pltpu.VMEM((D, BW), jnp.float32),
        pltpu.SemaphoreType.DMA,
        pltpu.SemaphoreType.DMA,
        pltpu.SemaphoreType.DMA,
        pltpu.SemaphoreType.DMA,
    ],
    compiler_params=pltpu.CompilerParams(
        use_tc_tiling_on_sc=False, needs_layout_passes=False),
)
def _gather_kernel(actionT_hbm, table_hbm, outT_hbm,
                   idx_v, rows0, rows1, rT0, rT1, sg0, sg1, sw0, sw1):
    wid = lax.axis_index("s") * NC + lax.axis_index("c")
    b0 = wid * BW
    rows = (rows0, rows1)
    rT = (rT0, rT1)
    sg = (sg0, sg1)
    sw = (sw0, sw1)

    iota16 = lax.iota(jnp.int32, 16)
    d_consts = [jnp.full((16,), d, jnp.int32) for d in range(D)]

    pltpu.sync_copy(actionT_hbm.at[:, pl.ds(b0, BW)], idx_v)

    pltpu.async_copy(table_hbm.at[idx_v.at[0]], rows0, sg0)
    pltpu.async_copy(table_hbm.at[idx_v.at[1]], rows1, sg1)

    def s_body(s2, _):
        for b in range(2):
            s = s2 * 2 + b
            pltpu.make_async_copy(
                table_hbm.at[idx_v.at[s]], rows[b], sg[b]).wait()
            @pl.when(s2 >= 1)
            def _():
                pltpu.make_async_copy(
                    rT[b], outT_hbm.at[s, :, pl.ds(b0, BW)], sw[b]).wait()

            def tr_body(g, _):
                r_idx = iota16 + g * 16
                for d in range(D):
                    v = plsc.load_gather(rows[b], [r_idx, d_consts[d]])
                    rT[b][d, pl.ds(g * 16, 16)] = v
                return 0

            lax.fori_loop(0, BW // 16, tr_body, 0)

            pltpu.async_copy(rT[b], outT_hbm.at[s, :, pl.ds(b0, BW)], sw[b])

            @pl.when(s + 2 < S)
            def _():
                pltpu.async_copy(
                    table_hbm.at[idx_v.at[s + 2]], rows[b], sg[b])
        return 0

    lax.fori_loop(0, S // 2, s_body, 0)

    pltpu.make_async_copy(rT0, outT_hbm.at[S - 2, :, pl.ds(b0, BW)], sw0).wait()
    pltpu.make_async_copy(rT1, outT_hbm.at[S - 1, :, pl.ds(b0, BW)], sw1).wait()


def kernel(action, emb_table):
    actionT = action.T
    outT = _gather_kernel(actionT, emb_table)
    return outT.transpose(2, 0, 1)


# speedup vs baseline: 2.2744x; 1.6814x over previous
import functools

import jax
import jax.numpy as jnp
from jax import lax
from jax.experimental import pallas as pl
from jax.experimental.pallas import tpu as pltpu
from jax.experimental.pallas import tpu_sc as plsc

E = 1000000
D = 32
BROWS = 16384
S = 50
NC, NS = 2, 16
NW = NC * NS
BW = BROWS // NW
BWP = BW + 1
UNROLL = 8

_mesh = plsc.VectorSubcoreMesh(core_axis_name="c", subcore_axis_name="s")


@functools.partial(
    pl.kernel,
    mesh=_mesh,
    out_type=jax.ShapeDtypeStruct((S, D, BROWS), jnp.float32),
    scratch_types=[
        pltpu.VMEM((S, BW), jnp.int32),
        pltpu.VMEM((BW, D), jnp.float32),
        pltpu.VMEM((BW, D), jnp.float32),
        pltpu.VMEM((D, BWP), jnp.float32),
        pltpu.VMEM((D, BWP), jnp.float32),
        pltpu.SemaphoreType.DMA,
        pltpu.SemaphoreType.DMA,
        pltpu.SemaphoreType.DMA,
        pltpu.SemaphoreType.DMA,
    ],
    compiler_params=pltpu.CompilerParams(
        use_tc_tiling_on_sc=False, needs_layout_passes=False),
)
def _gather_kernel(actionT_hbm, table_hbm, outT_hbm,
                   idx_v, rows0, rows1, rT0, rT1, sg0, sg1, sw0, sw1):
    wid = lax.axis_index("s") * NC + lax.axis_index("c")
    b0 = wid * BW
    rows = (rows0, rows1)
    rT = (rT0, rT1)
    sg = (sg0, sg1)
    sw = (sw0, sw1)

    d_lo = lax.iota(jnp.int32, 16)
    d_hi = d_lo + 16

    pltpu.sync_copy(actionT_hbm.at[:, pl.ds(b0, BW)], idx_v)

    pltpu.async_copy(table_hbm.at[idx_v.at[0]], rows0, sg0)
    pltpu.async_copy(table_hbm.at[idx_v.at[1]], rows1, sg1)

    def s_body(s2, _):
        for b in range(2):
            s = s2 * 2 + b
            pltpu.make_async_copy(
                table_hbm.at[idx_v.at[s]], rows[b], sg[b]).wait()
            @pl.when(s2 >= 1)
            def _():
                pltpu.make_async_copy(
                    rT[b].at[:, pl.ds(0, BW)],
                    outT_hbm.at[s, :, pl.ds(b0, BW)], sw[b]).wait()

            def tr_body(g, _):
                for u in range(UNROLL):
                    r = g * UNROLL + u
                    r_vec = jnp.full((16,), 0, jnp.int32) + r
                    lo = rows[b][r, pl.ds(0, 16)]
                    hi = rows[b][r, pl.ds(16, 16)]
                    plsc.store_scatter(rT[b], [d_lo, r_vec], lo)
                    plsc.store_scatter(rT[b], [d_hi, r_vec], hi)
                return 0

            lax.fori_loop(0, BW // UNROLL, tr_body, 0)

            pltpu.async_copy(
                rT[b].at[:, pl.ds(0, BW)],
                outT_hbm.at[s, :, pl.ds(b0, BW)], sw[b])

            @pl.when(s + 2 < S)
            def _():
                pltpu.async_copy(
                    table_hbm.at[idx_v.at[s + 2]], rows[b], sg[b])
        return 0

    lax.fori_loop(0, S // 2, s_body, 0)

    pltpu.make_async_copy(
        rT0.at[:, pl.ds(0, BW)],
        outT_hbm.at[S - 2, :, pl.ds(b0, BW)], sw0).wait()
    pltpu.make_async_copy(
        rT1.at[:, pl.ds(0, BW)],
        outT_hbm.at[S - 1, :, pl.ds(b0, BW)], sw1).wait()


def kernel(action, emb_table):
    actionT = action.T
    outT = _gather_kernel(actionT, emb_table)
    return outT.transpose(2, 0, 1)
